# baseline (speedup 1.0000x reference)
import functools

import jax
import jax.numpy as jnp
from jax.experimental import pallas as pl

N = 2048
D = 768
E = 8
CAP = 512
HEADS = 12
HD = 64
MLP = 3072
F1 = 5376
F2IN = 3840
F2OUT = 1536
F1_BLK = 1792
SCALE = HD ** -0.5


def _router_kernel(x_ref, wr_ref, g_ref, b_ref, l_ref, y_ref, selp_ref, rank_ref):
    x = x_ref[...]
    m = jnp.mean(x, axis=1, keepdims=True)
    xc = x - m
    v = jnp.mean(xc * xc, axis=1, keepdims=True)
    y = xc * jax.lax.rsqrt(v + 1e-5) * g_ref[...] + b_ref[...]
    y_ref[...] = y.astype(jnp.bfloat16)

    lt = jax.lax.dot_general(wr_ref[...], x, (((0,), (1,)), ((), ())),
                             preferred_element_type=jnp.float32)
    mx = jnp.max(lt, axis=0, keepdims=True)
    ex = jnp.exp(lt - mx)
    pt = ex / jnp.sum(ex, axis=0, keepdims=True)

    pti = jax.lax.bitcast_convert_type(pt, jnp.int32)

    def bs_body(_, carry):
        lo, hi = carry
        mid = jax.lax.shift_right_logical(lo + hi, 1)
        cnt = jnp.sum((pti >= mid).astype(jnp.float32), axis=1, keepdims=True)
        big = cnt >= CAP
        return jnp.where(big, mid, lo), jnp.where(big, hi, mid)

    lo0 = jnp.zeros((E, 1), jnp.int32)
    hi0 = jnp.full((E, 1), 0x3F800001, jnp.int32)
    vstar, _ = jax.lax.fori_loop(0, 31, bs_body, (lo0, hi0))
    n_gt = jnp.sum((pti > vstar).astype(jnp.float32), axis=1, keepdims=True)
    ties_needed = CAP - n_gt

    pt_t = pt.T
    pti_t = jax.lax.bitcast_convert_type(pt_t, jnp.int32)
    vst = vstar.T
    tneed = ties_needed.T
    gt = pti_t > vst
    eq = pti_t == vst
    both = jnp.concatenate([gt.astype(jnp.bfloat16), eq.astype(jnp.bfloat16)],
                           axis=1)
    ranks = jnp.dot(l_ref[...], both, preferred_element_type=jnp.float32)
    gt_rank = ranks[:, :E]
    tie_rank = ranks[:, E:]
    sel = gt | (eq & (tie_rank < tneed))
    sel_rank = gt_rank + jnp.minimum(tie_rank, tneed)
    selp_ref[...] = jnp.where(sel, pt_t, 0.0)
    rank_ref[...] = jnp.where(sel, sel_rank, -1.0)


def _lane_pick(arr, e):
    lane = jax.lax.broadcasted_iota(jnp.int32, (N, E), 1)
    return jnp.sum(jnp.where(lane == e, arr, 0.0), axis=1, keepdims=True)


def _dispatch_kernel(selp_ref, rank_ref, y_ref, pt_ref, xe_ref):
    e = pl.program_id(0)
    re = _lane_pick(rank_ref[...], e)
    lane_c = jax.lax.broadcasted_iota(jnp.int32, (N, CAP), 1).astype(jnp.float32)
    p_t = (re == lane_c).astype(jnp.bfloat16)
    pt_ref[0] = p_t
    xe = jax.lax.dot_general(p_t, y_ref[...], (((0,), (0,)), ((), ())),
                             preferred_element_type=jnp.float32)
    xe_ref[0] = xe.astype(jnp.bfloat16)


def _fc1_kernel(xe_ref, w1_ref, b1_ref, pt_ref, out_ref):
    e = pl.program_id(1)
    xe = xe_ref[0]
    w = w1_ref[0].astype(jnp.bfloat16)
    h = jnp.dot(xe, w, preferred_element_type=jnp.float32)
    h = h + b1_ref[...]
    hb = h.astype(jnp.bfloat16)
    contrib = jnp.dot(pt_ref[0], hb, preferred_element_type=jnp.float32)

    @pl.when(e == 0)
    def _():
        out_ref[...] = contrib

    @pl.when(e != 0)
    def _():
        out_ref[...] += contrib


def _attn_kernel(q_ref, k_ref, v_ref, xa_ref):
    qb = q_ref[...].astype(jnp.bfloat16)
    kb = k_ref[...].astype(jnp.bfloat16)
    vb = v_ref[...].astype(jnp.bfloat16)
    for j in range(2):
        q = qb[:, j * HD:(j + 1) * HD]
        k = kb[:, j * HD:(j + 1) * HD]
        v = vb[:, j * HD:(j + 1) * HD]
        s = jax.lax.dot_general(q, k, (((1,), (1,)), ((), ())),
                                preferred_element_type=jnp.float32) * SCALE
        smax = jnp.max(s, axis=1, keepdims=True)
        p = jnp.exp(s - smax)
        p = p / jnp.sum(p, axis=1, keepdims=True)
        o = jnp.dot(p.astype(jnp.bfloat16), v,
                    preferred_element_type=jnp.float32)
        xa_ref[:, j * HD:(j + 1) * HD] = o.astype(jnp.bfloat16)


def _gelu_kernel(h_ref, o_ref):
    h = h_ref[...]
    o_ref[...] = (0.5 * h * (1.0 + jax.lax.erf(h * (2.0 ** -0.5)))).astype(jnp.bfloat16)


def _gather2_kernel(pt_ref, ym_ref, xa_ref, ye_ref):
    p_t = pt_ref[0]
    yem = jax.lax.dot_general(p_t, ym_ref[...], (((0,), (0,)), ((), ())),
                              preferred_element_type=jnp.float32)
    yea = jax.lax.dot_general(p_t, xa_ref[...], (((0,), (0,)), ((), ())),
                              preferred_element_type=jnp.float32)
    ye_ref[0, :, :MLP] = yem.astype(jnp.bfloat16)
    ye_ref[0, :, MLP:] = yea.astype(jnp.bfloat16)


def _fc2_kernel(ye_ref, w2_ref, b2_ref, selp_ref, pt_ref, out_ref):
    e = pl.program_id(1)
    ye = ye_ref[0]
    w = w2_ref[0].astype(jnp.bfloat16)
    o = jnp.dot(ye, w, preferred_element_type=jnp.float32)
    ob = (o + b2_ref[...]).astype(jnp.bfloat16)
    pe = _lane_pick(selp_ref[...], e)
    contrib = pe * jnp.dot(pt_ref[0], ob, preferred_element_type=jnp.float32)

    @pl.when(e == 0)
    def _():
        out_ref[...] = contrib

    @pl.when(e != 0)
    def _():
        out_ref[...] += contrib


def _final_kernel(x_ref, ot_ref, o_ref):
    o_ref[...] = x_ref[...] + ot_ref[:, :D] + ot_ref[:, D:]


def kernel(x, Wr, norm_g, norm_b, W1, b1, W2, b2):
    x2 = x[0]
    g2 = norm_g.reshape(1, D)
    b2n = norm_b.reshape(1, D)
    b1r = b1.reshape(1, F1)
    b2r = b2.reshape(1, F2OUT)

    tri = jnp.tril(jnp.ones((N, N), jnp.bfloat16), -1)

    y, selp, rank = pl.pallas_call(
        _router_kernel,
        out_shape=(
            jax.ShapeDtypeStruct((N, D), jnp.bfloat16),
            jax.ShapeDtypeStruct((N, E), jnp.float32),
            jax.ShapeDtypeStruct((N, E), jnp.float32),
        ),
    )(x2, Wr, g2, b2n, tri)

    pt_all, xe_all = pl.pallas_call(
        _dispatch_kernel,
        grid=(E,),
        in_specs=[
            pl.BlockSpec((N, E), lambda e: (0, 0)),
            pl.BlockSpec((N, E), lambda e: (0, 0)),
            pl.BlockSpec((N, D), lambda e: (0, 0)),
        ],
        out_specs=(
            pl.BlockSpec((1, N, CAP), lambda e: (e, 0, 0)),
            pl.BlockSpec((1, CAP, D), lambda e: (e, 0, 0)),
        ),
        out_shape=(
            jax.ShapeDtypeStruct((E, N, CAP), jnp.bfloat16),
            jax.ShapeDtypeStruct((E, CAP, D), jnp.bfloat16),
        ),
    )(selp, rank, y)

    h_tok = pl.pallas_call(
        _fc1_kernel,
        grid=(F1 // F1_BLK, E),
        in_specs=[
            pl.BlockSpec((1, CAP, D), lambda f, e: (e, 0, 0)),
            pl.BlockSpec((1, D, F1_BLK), lambda f, e: (e, 0, f)),
            pl.BlockSpec((1, F1_BLK), lambda f, e: (0, f)),
            pl.BlockSpec((1, N, CAP), lambda f, e: (e, 0, 0)),
        ],
        out_specs=pl.BlockSpec((N, F1_BLK), lambda f, e: (0, f)),
        out_shape=jax.ShapeDtypeStruct((N, F1), jnp.float32),
    )(xe_all, W1, b1r, pt_all)

    xa = pl.pallas_call(
        _attn_kernel,
        grid=(HEADS // 2,),
        in_specs=[
            pl.BlockSpec((N, 2 * HD), lambda h: (0, (MLP // (2 * HD)) + h)),
            pl.BlockSpec((N, 2 * HD), lambda h: (0, ((MLP + D) // (2 * HD)) + h)),
            pl.BlockSpec((N, 2 * HD), lambda h: (0, ((MLP + 2 * D) // (2 * HD)) + h)),
        ],
        out_specs=pl.BlockSpec((N, 2 * HD), lambda h: (0, h)),
        out_shape=jax.ShapeDtypeStruct((N, D), jnp.bfloat16),
    )(h_tok, h_tok, h_tok)

    y2m = pl.pallas_call(
        _gelu_kernel,
        grid=(4,),
        in_specs=[pl.BlockSpec((N, MLP // 4), lambda i: (0, i))],
        out_specs=pl.BlockSpec((N, MLP // 4), lambda i: (0, i)),
        out_shape=jax.ShapeDtypeStruct((N, MLP), jnp.bfloat16),
    )(h_tok)

    ye_all = pl.pallas_call(
        _gather2_kernel,
        grid=(E,),
        in_specs=[
            pl.BlockSpec((1, N, CAP), lambda e: (e, 0, 0)),
            pl.BlockSpec((N, MLP), lambda e: (0, 0)),
            pl.BlockSpec((N, D), lambda e: (0, 0)),
        ],
        out_specs=pl.BlockSpec((1, CAP, F2IN), lambda e: (e, 0, 0)),
        out_shape=jax.ShapeDtypeStruct((E, CAP, F2IN), jnp.bfloat16),
    )(pt_all, y2m, xa)

    out_tok = pl.pallas_call(
        _fc2_kernel,
        grid=(2, E),
        in_specs=[
            pl.BlockSpec((1, CAP, F2IN), lambda o, e: (e, 0, 0)),
            pl.BlockSpec((1, F2IN, F2OUT // 2), lambda o, e: (e, 0, o)),
            pl.BlockSpec((1, F2OUT // 2), lambda o, e: (0, o)),
            pl.BlockSpec((N, E), lambda o, e: (0, 0)),
            pl.BlockSpec((1, N, CAP), lambda o, e: (e, 0, 0)),
        ],
        out_specs=pl.BlockSpec((N, F2OUT // 2), lambda o, e: (0, o)),
        out_shape=jax.ShapeDtypeStruct((N, F2OUT), jnp.float32),
    )(ye_all, W2, b2r, selp, pt_all)

    out = pl.pallas_call(
        _final_kernel,
        in_specs=[
            pl.BlockSpec((N, D), lambda: (0, 0)),
            pl.BlockSpec((N, F2OUT), lambda: (0, 0)),
        ],
        out_specs=pl.BlockSpec((N, D), lambda: (0, 0)),
        out_shape=jax.ShapeDtypeStruct((N, D), jnp.float32),
    )(x2, out_tok)

    return out[None]


# speedup vs baseline: 1.0115x; 1.0115x over previous
import functools

import jax
import jax.numpy as jnp
from jax.experimental import pallas as pl
from jax.experimental.pallas import tpu as pltpu

N = 2048
D = 768
E = 8
CAP = 512
HEADS = 12
HD = 64
MLP = 3072
F1 = 5376
F2IN = 3840
F2OUT = 1536
F1_BLK = 1792
SCALE = HD ** -0.5


def _router_kernel(x_ref, wr_ref, g_ref, b_ref, l_ref, y_ref, selp_ref, rank_ref):
    x = x_ref[...]
    m = jnp.mean(x, axis=1, keepdims=True)
    xc = x - m
    v = jnp.mean(xc * xc, axis=1, keepdims=True)
    y = xc * jax.lax.rsqrt(v + 1e-5) * g_ref[...] + b_ref[...]
    y_ref[...] = y.astype(jnp.bfloat16)

    lt = jax.lax.dot_general(wr_ref[...], x, (((0,), (1,)), ((), ())),
                             preferred_element_type=jnp.float32)
    mx = jnp.max(lt, axis=0, keepdims=True)
    ex = jnp.exp(lt - mx)
    pt = ex / jnp.sum(ex, axis=0, keepdims=True)

    pti = jax.lax.bitcast_convert_type(pt, jnp.int32)

    def bs_body(_, carry):
        lo, hi = carry
        mid = jax.lax.shift_right_logical(lo + hi, 1)
        cnt = jnp.sum((pti >= mid).astype(jnp.float32), axis=1, keepdims=True)
        big = cnt >= CAP
        return jnp.where(big, mid, lo), jnp.where(big, hi, mid)

    lo0 = jnp.zeros((E, 1), jnp.int32)
    hi0 = jnp.full((E, 1), 0x3F800001, jnp.int32)
    vstar, _ = jax.lax.fori_loop(0, 31, bs_body, (lo0, hi0))
    n_gt = jnp.sum((pti > vstar).astype(jnp.float32), axis=1, keepdims=True)
    ties_needed = CAP - n_gt

    pt_t = pt.T
    pti_t = jax.lax.bitcast_convert_type(pt_t, jnp.int32)
    vst = vstar.T
    tneed = ties_needed.T
    gt = pti_t > vst
    eq = pti_t == vst
    both = jnp.concatenate([gt.astype(jnp.bfloat16), eq.astype(jnp.bfloat16)],
                           axis=1)
    ranks = jnp.dot(l_ref[...], both, preferred_element_type=jnp.float32)
    gt_rank = ranks[:, :E]
    tie_rank = ranks[:, E:]
    sel = gt | (eq & (tie_rank < tneed))
    sel_rank = gt_rank + jnp.minimum(tie_rank, tneed)
    selp_ref[...] = jnp.where(sel, pt_t, 0.0)
    rank_ref[...] = jnp.where(sel, sel_rank, -1.0)


def _lane_pick(arr, e):
    lane = jax.lax.broadcasted_iota(jnp.int32, (N, E), 1)
    return jnp.sum(jnp.where(lane == e, arr, 0.0), axis=1, keepdims=True)


def _dispatch_kernel(selp_ref, rank_ref, y_ref, pt_ref, xe_ref):
    e = pl.program_id(0)
    re = _lane_pick(rank_ref[...], e)
    lane_c = jax.lax.broadcasted_iota(jnp.int32, (N, CAP), 1).astype(jnp.float32)
    p_t = (re == lane_c).astype(jnp.bfloat16)
    pt_ref[0] = p_t
    xe = jax.lax.dot_general(p_t, y_ref[...], (((0,), (0,)), ((), ())),
                             preferred_element_type=jnp.float32)
    xe_ref[0] = xe.astype(jnp.bfloat16)


def _gelu(h):
    return 0.5 * h * (1.0 + jax.lax.erf(h * (2.0 ** -0.5)))


def _fc1_kernel(xe_ref, w1_ref, b1_ref, pt_ref, out_ref, acc_ref):
    f = pl.program_id(0)
    e = pl.program_id(1)
    xe = xe_ref[0]
    w = w1_ref[0].astype(jnp.bfloat16)
    h = jnp.dot(xe, w, preferred_element_type=jnp.float32)
    h = h + b1_ref[...]
    hb = h.astype(jnp.bfloat16)
    contrib = jnp.dot(pt_ref[0], hb, preferred_element_type=jnp.float32)

    @pl.when(e == 0)
    def _():
        acc_ref[...] = contrib

    @pl.when(e != 0)
    def _():
        acc_ref[...] += contrib

    @pl.when((e == E - 1) & (f == 0))
    def _():
        out_ref[...] = _gelu(acc_ref[...]).astype(jnp.bfloat16)

    @pl.when((e == E - 1) & (f == 1))
    def _():
        acc = acc_ref[...]
        col = jax.lax.broadcasted_iota(jnp.int32, (N, F1_BLK), 1) + F1_BLK
        out_ref[...] = jnp.where(col < MLP, _gelu(acc), acc).astype(jnp.bfloat16)

    @pl.when((e == E - 1) & (f == 2))
    def _():
        out_ref[...] = acc_ref[...].astype(jnp.bfloat16)


def _attn_kernel(q_ref, k_ref, v_ref, xa_ref):
    qb = q_ref[...].astype(jnp.bfloat16)
    kb = k_ref[...].astype(jnp.bfloat16)
    vb = v_ref[...].astype(jnp.bfloat16)
    for j in range(2):
        q = qb[:, j * HD:(j + 1) * HD]
        k = kb[:, j * HD:(j + 1) * HD]
        v = vb[:, j * HD:(j + 1) * HD]
        s = jax.lax.dot_general(q, k, (((1,), (1,)), ((), ())),
                                preferred_element_type=jnp.float32) * SCALE
        smax = jnp.max(s, axis=1, keepdims=True)
        p = jnp.exp(s - smax)
        p = p / jnp.sum(p, axis=1, keepdims=True)
        o = jnp.dot(p.astype(jnp.bfloat16), v,
                    preferred_element_type=jnp.float32)
        xa_ref[:, j * HD:(j + 1) * HD] = o.astype(jnp.bfloat16)


def _gather2_kernel(pt_ref, ym_ref, xa_ref, ye_ref):
    p_t = pt_ref[0]
    yem = jax.lax.dot_general(p_t, ym_ref[...], (((0,), (0,)), ((), ())),
                              preferred_element_type=jnp.float32)
    yea = jax.lax.dot_general(p_t, xa_ref[...], (((0,), (0,)), ((), ())),
                              preferred_element_type=jnp.float32)
    ye_ref[0, :, :MLP] = yem.astype(jnp.bfloat16)
    ye_ref[0, :, MLP:] = yea.astype(jnp.bfloat16)


def _fc2_kernel(ye_ref, w2_ref, b2_ref, selp_ref, pt_ref, out_ref):
    e = pl.program_id(1)
    ye = ye_ref[0]
    w = w2_ref[0].astype(jnp.bfloat16)
    o = jnp.dot(ye, w, preferred_element_type=jnp.float32)
    ob = (o + b2_ref[...]).astype(jnp.bfloat16)
    pe = _lane_pick(selp_ref[...], e)
    contrib = pe * jnp.dot(pt_ref[0], ob, preferred_element_type=jnp.float32)

    @pl.when(e == 0)
    def _():
        out_ref[...] = contrib

    @pl.when(e != 0)
    def _():
        out_ref[...] += contrib


def _final_kernel(x_ref, ot_ref, o_ref):
    o_ref[...] = x_ref[...] + ot_ref[:, :D] + ot_ref[:, D:]


def kernel(x, Wr, norm_g, norm_b, W1, b1, W2, b2):
    x2 = x[0]
    g2 = norm_g.reshape(1, D)
    b2n = norm_b.reshape(1, D)
    b1r = b1.reshape(1, F1)
    b2r = b2.reshape(1, F2OUT)

    tri = jnp.tril(jnp.ones((N, N), jnp.bfloat16), -1)

    y, selp, rank = pl.pallas_call(
        _router_kernel,
        out_shape=(
            jax.ShapeDtypeStruct((N, D), jnp.bfloat16),
            jax.ShapeDtypeStruct((N, E), jnp.float32),
            jax.ShapeDtypeStruct((N, E), jnp.float32),
        ),
    )(x2, Wr, g2, b2n, tri)

    pt_all, xe_all = pl.pallas_call(
        _dispatch_kernel,
        grid=(E,),
        in_specs=[
            pl.BlockSpec((N, E), lambda e: (0, 0)),
            pl.BlockSpec((N, E), lambda e: (0, 0)),
            pl.BlockSpec((N, D), lambda e: (0, 0)),
        ],
        out_specs=(
            pl.BlockSpec((1, N, CAP), lambda e: (e, 0, 0)),
            pl.BlockSpec((1, CAP, D), lambda e: (e, 0, 0)),
        ),
        out_shape=(
            jax.ShapeDtypeStruct((E, N, CAP), jnp.bfloat16),
            jax.ShapeDtypeStruct((E, CAP, D), jnp.bfloat16),
        ),
    )(selp, rank, y)

    h_tok = pl.pallas_call(
        _fc1_kernel,
        grid=(F1 // F1_BLK, E),
        in_specs=[
            pl.BlockSpec((1, CAP, D), lambda f, e: (e, 0, 0)),
            pl.BlockSpec((1, D, F1_BLK), lambda f, e: (e, 0, f)),
            pl.BlockSpec((1, F1_BLK), lambda f, e: (0, f)),
            pl.BlockSpec((1, N, CAP), lambda f, e: (e, 0, 0)),
        ],
        out_specs=pl.BlockSpec((N, F1_BLK), lambda f, e: (0, f)),
        out_shape=jax.ShapeDtypeStruct((N, F1), jnp.bfloat16),
        scratch_shapes=[pltpu.VMEM((N, F1_BLK), jnp.float32)],
    )(xe_all, W1, b1r, pt_all)

    xa = pl.pallas_call(
        _attn_kernel,
        grid=(HEADS // 2,),
        in_specs=[
            pl.BlockSpec((N, 2 * HD), lambda h: (0, (MLP // (2 * HD)) + h)),
            pl.BlockSpec((N, 2 * HD), lambda h: (0, ((MLP + D) // (2 * HD)) + h)),
            pl.BlockSpec((N, 2 * HD), lambda h: (0, ((MLP + 2 * D) // (2 * HD)) + h)),
        ],
        out_specs=pl.BlockSpec((N, 2 * HD), lambda h: (0, h)),
        out_shape=jax.ShapeDtypeStruct((N, D), jnp.bfloat16),
    )(h_tok, h_tok, h_tok)

    ye_all = pl.pallas_call(
        _gather2_kernel,
        grid=(E,),
        in_specs=[
            pl.BlockSpec((1, N, CAP), lambda e: (e, 0, 0)),
            pl.BlockSpec((N, MLP), lambda e: (0, 0)),
            pl.BlockSpec((N, D), lambda e: (0, 0)),
        ],
        out_specs=pl.BlockSpec((1, CAP, F2IN), lambda e: (e, 0, 0)),
        out_shape=jax.ShapeDtypeStruct((E, CAP, F2IN), jnp.bfloat16),
    )(pt_all, h_tok, xa)

    out_tok = pl.pallas_call(
        _fc2_kernel,
        grid=(2, E),
        in_specs=[
            pl.BlockSpec((1, CAP, F2IN), lambda o, e: (e, 0, 0)),
            pl.BlockSpec((1, F2IN, F2OUT // 2), lambda o, e: (e, 0, o)),
            pl.BlockSpec((1, F2OUT // 2), lambda o, e: (0, o)),
            pl.BlockSpec((N, E), lambda o, e: (0, 0)),
            pl.BlockSpec((1, N, CAP), lambda o, e: (e, 0, 0)),
        ],
        out_specs=pl.BlockSpec((N, F2OUT // 2), lambda o, e: (0, o)),
        out_shape=jax.ShapeDtypeStruct((N, F2OUT), jnp.float32),
    )(ye_all, W2, b2r, selp, pt_all)

    out = pl.pallas_call(
        _final_kernel,
        in_specs=[
            pl.BlockSpec((N, D), lambda: (0, 0)),
            pl.BlockSpec((N, F2OUT), lambda: (0, 0)),
        ],
        out_specs=pl.BlockSpec((N, D), lambda: (0, 0)),
        out_shape=jax.ShapeDtypeStruct((N, D), jnp.float32),
    )(x2, out_tok)

    return out[None]


# speedup vs baseline: 1.0294x; 1.0177x over previous
import functools

import jax
import jax.numpy as jnp
from jax.experimental import pallas as pl
from jax.experimental.pallas import tpu as pltpu

N = 2048
D = 768
E = 8
CAP = 512
HEADS = 12
HD = 64
MLP = 3072
F1 = 5376
F2IN = 3840
F2OUT = 1536
F1_BLK = 1792
SCALE = HD ** -0.5


def _router_kernel(x_ref, wr_ref, g_ref, b_ref, l_ref, y_ref, selp_ref, rank_ref):
    x = x_ref[...]
    m = jnp.mean(x, axis=1, keepdims=True)
    xc = x - m
    v = jnp.mean(xc * xc, axis=1, keepdims=True)
    y = xc * jax.lax.rsqrt(v + 1e-5) * g_ref[...] + b_ref[...]
    y_ref[...] = y.astype(jnp.bfloat16)

    lt = jax.lax.dot_general(wr_ref[...], x, (((0,), (1,)), ((), ())),
                             preferred_element_type=jnp.float32)
    mx = jnp.max(lt, axis=0, keepdims=True)
    ex = jnp.exp(lt - mx)
    pt = ex / jnp.sum(ex, axis=0, keepdims=True)

    pti = jax.lax.bitcast_convert_type(pt, jnp.int32)

    def bs_body(_, carry):
        lo, hi = carry
        mid = jax.lax.shift_right_logical(lo + hi, 1)
        cnt = jnp.sum((pti >= mid).astype(jnp.float32), axis=1, keepdims=True)
        big = cnt >= CAP
        return jnp.where(big, mid, lo), jnp.where(big, hi, mid)

    lo0 = jnp.zeros((E, 1), jnp.int32)
    hi0 = jnp.full((E, 1), 0x3F800001, jnp.int32)
    vstar, _ = jax.lax.fori_loop(0, 31, bs_body, (lo0, hi0))
    n_gt = jnp.sum((pti > vstar).astype(jnp.float32), axis=1, keepdims=True)
    ties_needed = CAP - n_gt

    pt_t = pt.T
    pti_t = jax.lax.bitcast_convert_type(pt_t, jnp.int32)
    vst = vstar.T
    tneed = ties_needed.T
    gt = pti_t > vst
    eq = pti_t == vst
    both = jnp.concatenate([gt.astype(jnp.bfloat16), eq.astype(jnp.bfloat16)],
                           axis=1)
    ranks = jnp.dot(l_ref[...], both, preferred_element_type=jnp.float32)
    gt_rank = ranks[:, :E]
    tie_rank = ranks[:, E:]
    sel = gt | (eq & (tie_rank < tneed))
    sel_rank = gt_rank + jnp.minimum(tie_rank, tneed)
    selp_ref[...] = jnp.where(sel, pt_t, 0.0)
    rank_ref[...] = jnp.where(sel, sel_rank, -1.0)


def _lane_pick(arr, e):
    lane = jax.lax.broadcasted_iota(jnp.int32, (N, E), 1)
    return jnp.sum(jnp.where(lane == e, arr, 0.0), axis=1, keepdims=True)


def _dispatch_kernel(selp_ref, rank_ref, y_ref, pt_ref, xe_ref):
    e = pl.program_id(0)
    re = _lane_pick(rank_ref[...], e)
    lane_c = jax.lax.broadcasted_iota(jnp.int32, (N, CAP), 1).astype(jnp.float32)
    p_t = (re == lane_c).astype(jnp.bfloat16)
    pt_ref[0] = p_t
    xe = jax.lax.dot_general(p_t, y_ref[...], (((0,), (0,)), ((), ())),
                             preferred_element_type=jnp.float32)
    xe_ref[0] = xe.astype(jnp.bfloat16)


def _gelu(h):
    return 0.5 * h * (1.0 + jax.lax.erf(h * (2.0 ** -0.5)))


def _fc1_kernel(xe_ref, w1_ref, b1_ref, pt_ref, out_ref, acc_ref):
    f = pl.program_id(0)
    e = pl.program_id(1)
    xe = xe_ref[0]
    w = w1_ref[0].astype(jnp.bfloat16)
    h = jnp.dot(xe, w, preferred_element_type=jnp.float32)
    h = h + b1_ref[...]
    hb = h.astype(jnp.bfloat16)
    contrib = jnp.dot(pt_ref[0], hb, preferred_element_type=jnp.float32)

    @pl.when(e == 0)
    def _():
        acc_ref[...] = contrib

    @pl.when(e != 0)
    def _():
        acc_ref[...] += contrib

    @pl.when((e == E - 1) & (f == 0))
    def _():
        out_ref[...] = _gelu(acc_ref[...]).astype(jnp.bfloat16)

    @pl.when((e == E - 1) & (f == 1))
    def _():
        acc = acc_ref[...]
        col = jax.lax.broadcasted_iota(jnp.int32, (N, F1_BLK), 1) + F1_BLK
        out_ref[...] = jnp.where(col < MLP, _gelu(acc), acc).astype(jnp.bfloat16)

    @pl.when((e == E - 1) & (f == 2))
    def _():
        out_ref[...] = acc_ref[...].astype(jnp.bfloat16)


def _attn_kernel(q_ref, k_ref, v_ref, xa_ref):
    qb = q_ref[...].astype(jnp.bfloat16)
    kb = k_ref[...].astype(jnp.bfloat16)
    vb = v_ref[...].astype(jnp.bfloat16)
    for j in range(2):
        q = (qb[:, j * HD:(j + 1) * HD].astype(jnp.float32)
             * SCALE).astype(jnp.bfloat16)
        k = kb[:, j * HD:(j + 1) * HD]
        v = vb[:, j * HD:(j + 1) * HD]
        s = jax.lax.dot_general(q, k, (((1,), (1,)), ((), ())),
                                preferred_element_type=jnp.float32
                                ).astype(jnp.bfloat16)
        smax = jnp.max(s, axis=1, keepdims=True)
        p = jnp.exp(s - smax)
        p = p / jnp.sum(p, axis=1, keepdims=True, dtype=jnp.float32).astype(jnp.bfloat16)
        o = jnp.dot(p, v, preferred_element_type=jnp.float32)
        xa_ref[:, j * HD:(j + 1) * HD] = o.astype(jnp.bfloat16)


def _gather2_kernel(pt_ref, ym_ref, xa_ref, ye_ref):
    p_t = pt_ref[0]
    yem = jax.lax.dot_general(p_t, ym_ref[...], (((0,), (0,)), ((), ())),
                              preferred_element_type=jnp.float32)
    yea = jax.lax.dot_general(p_t, xa_ref[...], (((0,), (0,)), ((), ())),
                              preferred_element_type=jnp.float32)
    ye_ref[0, :, :MLP] = yem.astype(jnp.bfloat16)
    ye_ref[0, :, MLP:] = yea.astype(jnp.bfloat16)


def _fc2_kernel(ye_ref, w2_ref, b2_ref, selp_ref, pt_ref, out_ref):
    e = pl.program_id(1)
    ye = ye_ref[0]
    w = w2_ref[0].astype(jnp.bfloat16)
    o = jnp.dot(ye, w, preferred_element_type=jnp.float32)
    ob = (o + b2_ref[...]).astype(jnp.bfloat16)
    pe = _lane_pick(selp_ref[...], e)
    contrib = pe * jnp.dot(pt_ref[0], ob, preferred_element_type=jnp.float32)

    @pl.when(e == 0)
    def _():
        out_ref[...] = contrib

    @pl.when(e != 0)
    def _():
        out_ref[...] += contrib


def _final_kernel(x_ref, ot_ref, o_ref):
    o_ref[...] = x_ref[...] + ot_ref[:, :D] + ot_ref[:, D:]


def kernel(x, Wr, norm_g, norm_b, W1, b1, W2, b2):
    x2 = x[0]
    g2 = norm_g.reshape(1, D)
    b2n = norm_b.reshape(1, D)
    b1r = b1.reshape(1, F1)
    b2r = b2.reshape(1, F2OUT)

    tri = jnp.tril(jnp.ones((N, N), jnp.bfloat16), -1)

    y, selp, rank = pl.pallas_call(
        _router_kernel,
        out_shape=(
            jax.ShapeDtypeStruct((N, D), jnp.bfloat16),
            jax.ShapeDtypeStruct((N, E), jnp.float32),
            jax.ShapeDtypeStruct((N, E), jnp.float32),
        ),
    )(x2, Wr, g2, b2n, tri)

    pt_all, xe_all = pl.pallas_call(
        _dispatch_kernel,
        grid=(E,),
        in_specs=[
            pl.BlockSpec((N, E), lambda e: (0, 0)),
            pl.BlockSpec((N, E), lambda e: (0, 0)),
            pl.BlockSpec((N, D), lambda e: (0, 0)),
        ],
        out_specs=(
            pl.BlockSpec((1, N, CAP), lambda e: (e, 0, 0)),
            pl.BlockSpec((1, CAP, D), lambda e: (e, 0, 0)),
        ),
        out_shape=(
            jax.ShapeDtypeStruct((E, N, CAP), jnp.bfloat16),
            jax.ShapeDtypeStruct((E, CAP, D), jnp.bfloat16),
        ),
    )(selp, rank, y)

    h_tok = pl.pallas_call(
        _fc1_kernel,
        grid=(F1 // F1_BLK, E),
        in_specs=[
            pl.BlockSpec((1, CAP, D), lambda f, e: (e, 0, 0)),
            pl.BlockSpec((1, D, F1_BLK), lambda f, e: (e, 0, f)),
            pl.BlockSpec((1, F1_BLK), lambda f, e: (0, f)),
            pl.BlockSpec((1, N, CAP), lambda f, e: (e, 0, 0)),
        ],
        out_specs=pl.BlockSpec((N, F1_BLK), lambda f, e: (0, f)),
        out_shape=jax.ShapeDtypeStruct((N, F1), jnp.bfloat16),
        scratch_shapes=[pltpu.VMEM((N, F1_BLK), jnp.float32)],
    )(xe_all, W1, b1r, pt_all)

    xa = pl.pallas_call(
        _attn_kernel,
        grid=(HEADS // 2,),
        in_specs=[
            pl.BlockSpec((N, 2 * HD), lambda h: (0, (MLP // (2 * HD)) + h)),
            pl.BlockSpec((N, 2 * HD), lambda h: (0, ((MLP + D) // (2 * HD)) + h)),
            pl.BlockSpec((N, 2 * HD), lambda h: (0, ((MLP + 2 * D) // (2 * HD)) + h)),
        ],
        out_specs=pl.BlockSpec((N, 2 * HD), lambda h: (0, h)),
        out_shape=jax.ShapeDtypeStruct((N, D), jnp.bfloat16),
    )(h_tok, h_tok, h_tok)

    ye_all = pl.pallas_call(
        _gather2_kernel,
        grid=(E,),
        in_specs=[
            pl.BlockSpec((1, N, CAP), lambda e: (e, 0, 0)),
            pl.BlockSpec((N, MLP), lambda e: (0, 0)),
            pl.BlockSpec((N, D), lambda e: (0, 0)),
        ],
        out_specs=pl.BlockSpec((1, CAP, F2IN), lambda e: (e, 0, 0)),
        out_shape=jax.ShapeDtypeStruct((E, CAP, F2IN), jnp.bfloat16),
    )(pt_all, h_tok, xa)

    out_tok = pl.pallas_call(
        _fc2_kernel,
        grid=(2, E),
        in_specs=[
            pl.BlockSpec((1, CAP, F2IN), lambda o, e: (e, 0, 0)),
            pl.BlockSpec((1, F2IN, F2OUT // 2), lambda o, e: (e, 0, o)),
            pl.BlockSpec((1, F2OUT // 2), lambda o, e: (0, o)),
            pl.BlockSpec((N, E), lambda o, e: (0, 0)),
            pl.BlockSpec((1, N, CAP), lambda o, e: (e, 0, 0)),
        ],
        out_specs=pl.BlockSpec((N, F2OUT // 2), lambda o, e: (0, o)),
        out_shape=jax.ShapeDtypeStruct((N, F2OUT), jnp.float32),
    )(ye_all, W2, b2r, selp, pt_all)

    out = pl.pallas_call(
        _final_kernel,
        in_specs=[
            pl.BlockSpec((N, D), lambda: (0, 0)),
            pl.BlockSpec((N, F2OUT), lambda: (0, 0)),
        ],
        out_specs=pl.BlockSpec((N, D), lambda: (0, 0)),
        out_shape=jax.ShapeDtypeStruct((N, D), jnp.float32),
    )(x2, out_tok)

    return out[None]


# speedup vs baseline: 1.0313x; 1.0018x over previous
import jax
import jax.numpy as jnp
from jax.experimental import pallas as pl
from jax.experimental.pallas import tpu as pltpu

N = 2048
D = 768
E = 8
CAP = 512
HEADS = 12
HD = 64
MLP = 3072
F1 = 5376
F2IN = 3840
F2OUT = 1536
F1_BLK = 1792
SCALE = HD ** -0.5


def _router_kernel(x_ref, wr_ref, g_ref, b_ref, l_ref, y_ref, selp_ref, rank_ref):
    x = x_ref[...]
    m = jnp.mean(x, axis=1, keepdims=True)
    xc = x - m
    v = jnp.mean(xc * xc, axis=1, keepdims=True)
    y = xc * jax.lax.rsqrt(v + 1e-5) * g_ref[...] + b_ref[...]
    y_ref[...] = y.astype(jnp.bfloat16)

    lt = jax.lax.dot_general(wr_ref[...], x, (((0,), (1,)), ((), ())),
                             preferred_element_type=jnp.float32)
    mx = jnp.max(lt, axis=0, keepdims=True)
    ex = jnp.exp(lt - mx)
    pt = ex / jnp.sum(ex, axis=0, keepdims=True)

    pti = jax.lax.bitcast_convert_type(pt, jnp.int32)

    def bs_body(_, carry):
        lo, hi = carry
        mid = jax.lax.shift_right_logical(lo + hi, 1)
        cnt = jnp.sum((pti >= mid).astype(jnp.float32), axis=1, keepdims=True)
        big = cnt >= CAP
        return jnp.where(big, mid, lo), jnp.where(big, hi, mid)

    lo0 = jnp.zeros((E, 1), jnp.int32)
    hi0 = jnp.full((E, 1), 0x3F800001, jnp.int32)
    vstar, _ = jax.lax.fori_loop(0, 31, bs_body, (lo0, hi0))
    n_gt = jnp.sum((pti > vstar).astype(jnp.float32), axis=1, keepdims=True)
    ties_needed = CAP - n_gt

    pt_t = pt.T
    pti_t = jax.lax.bitcast_convert_type(pt_t, jnp.int32)
    vst = vstar.T
    tneed = ties_needed.T
    gt = pti_t > vst
    eq = pti_t == vst
    both = jnp.concatenate([gt.astype(jnp.bfloat16), eq.astype(jnp.bfloat16)],
                           axis=1)
    ranks = jnp.dot(l_ref[...], both, preferred_element_type=jnp.float32)
    gt_rank = ranks[:, :E]
    tie_rank = ranks[:, E:]
    sel = gt | (eq & (tie_rank < tneed))
    sel_rank = gt_rank + jnp.minimum(tie_rank, tneed)
    selp_ref[...] = jnp.where(sel, pt_t, 0.0)
    rank_ref[...] = jnp.where(sel, sel_rank, -1.0)


def _lane_pick(arr, e):
    lane = jax.lax.broadcasted_iota(jnp.int32, (N, E), 1)
    return jnp.sum(jnp.where(lane == e, arr, 0.0), axis=1, keepdims=True)


def _dispatch_kernel(selp_ref, rank_ref, y_ref, pt_ref, xe_ref):
    e = pl.program_id(0)
    re = _lane_pick(rank_ref[...], e)
    lane_c = jax.lax.broadcasted_iota(jnp.int32, (N, CAP), 1).astype(jnp.float32)
    p_t = (re == lane_c).astype(jnp.bfloat16)
    pt_ref[0] = p_t
    xe = jax.lax.dot_general(p_t, y_ref[...], (((0,), (0,)), ((), ())),
                             preferred_element_type=jnp.float32)
    xe_ref[0] = xe.astype(jnp.bfloat16)


def _gelu(h):
    return 0.5 * h * (1.0 + jax.lax.erf(h * (2.0 ** -0.5)))


def _fc1_kernel(xe_ref, w1_ref, b1_ref, pt_ref, out_ref, acc_ref):
    f = pl.program_id(0)
    e = pl.program_id(1)
    xe = xe_ref[0]
    w = w1_ref[0].astype(jnp.bfloat16)
    h = jnp.dot(xe, w, preferred_element_type=jnp.float32)
    h = h + b1_ref[...]
    hb = h.astype(jnp.bfloat16)
    contrib = jnp.dot(pt_ref[0], hb, preferred_element_type=jnp.float32)

    @pl.when(e == 0)
    def _():
        acc_ref[...] = contrib

    @pl.when(e != 0)
    def _():
        acc_ref[...] += contrib

    @pl.when((e == E - 1) & (f == 0))
    def _():
        out_ref[...] = _gelu(acc_ref[...]).astype(jnp.bfloat16)

    @pl.when((e == E - 1) & (f == 1))
    def _():
        acc = acc_ref[...]
        col = jax.lax.broadcasted_iota(jnp.int32, (N, F1_BLK), 1) + F1_BLK
        out_ref[...] = jnp.where(col < MLP, _gelu(acc), acc).astype(jnp.bfloat16)

    @pl.when((e == E - 1) & (f == 2))
    def _():
        out_ref[...] = acc_ref[...].astype(jnp.bfloat16)


def _attn_kernel(q_ref, k_ref, v_ref, xa_ref):
    qb = q_ref[...].astype(jnp.bfloat16)
    kb = k_ref[...].astype(jnp.bfloat16)
    vb = v_ref[...].astype(jnp.bfloat16)
    for j in range(2):
        q = (qb[:, j * HD:(j + 1) * HD].astype(jnp.float32)
             * SCALE).astype(jnp.bfloat16)
        k = kb[:, j * HD:(j + 1) * HD]
        v = vb[:, j * HD:(j + 1) * HD]
        s = jax.lax.dot_general(q, k, (((1,), (1,)), ((), ())),
                                preferred_element_type=jnp.float32
                                ).astype(jnp.bfloat16)
        smax = jnp.max(s, axis=1, keepdims=True)
        p = jnp.exp(s - smax)
        p = p / jnp.sum(p, axis=1, keepdims=True, dtype=jnp.float32).astype(jnp.bfloat16)
        o = jnp.dot(p, v, preferred_element_type=jnp.float32)
        xa_ref[:, j * HD:(j + 1) * HD] = o.astype(jnp.bfloat16)


def _gather2_kernel(pt_ref, ym_ref, xa_ref, ye_ref):
    p_t = pt_ref[0]
    yem = jax.lax.dot_general(p_t, ym_ref[...], (((0,), (0,)), ((), ())),
                              preferred_element_type=jnp.float32)
    yea = jax.lax.dot_general(p_t, xa_ref[...], (((0,), (0,)), ((), ())),
                              preferred_element_type=jnp.float32)
    ye_ref[0, :, :MLP] = yem.astype(jnp.bfloat16)
    ye_ref[0, :, MLP:] = yea.astype(jnp.bfloat16)


def _fc2_kernel(ye_ref, w2_ref, b2_ref, selp_ref, pt_ref, out_ref):
    e = pl.program_id(1)
    ye = ye_ref[0]
    w = w2_ref[0].astype(jnp.bfloat16)
    o = jnp.dot(ye, w, preferred_element_type=jnp.float32)
    ob = (o + b2_ref[...]).astype(jnp.bfloat16)
    pe = _lane_pick(selp_ref[...], e)
    contrib = pe * jnp.dot(pt_ref[0], ob, preferred_element_type=jnp.float32)

    @pl.when(e == 0)
    def _():
        out_ref[...] = contrib

    @pl.when(e != 0)
    def _():
        out_ref[...] += contrib


def _final_kernel(x_ref, ot_ref, o_ref):
    o_ref[...] = x_ref[...] + ot_ref[:, :D] + ot_ref[:, D:]


def kernel(x, Wr, norm_g, norm_b, W1, b1, W2, b2):
    x2 = x[0]
    g2 = norm_g.reshape(1, D)
    b2n = norm_b.reshape(1, D)
    b1r = b1.reshape(1, F1)
    b2r = b2.reshape(1, F2OUT)

    tri = jnp.tril(jnp.ones((N, N), jnp.bfloat16), -1)

    y, selp, rank = pl.pallas_call(
        _router_kernel,
        out_shape=(
            jax.ShapeDtypeStruct((N, D), jnp.bfloat16),
            jax.ShapeDtypeStruct((N, E), jnp.float32),
            jax.ShapeDtypeStruct((N, E), jnp.float32),
        ),
    )(x2, Wr, g2, b2n, tri)

    pt_all, xe_all = pl.pallas_call(
        _dispatch_kernel,
        grid=(E,),
        in_specs=[
            pl.BlockSpec((N, E), lambda e: (0, 0)),
            pl.BlockSpec((N, E), lambda e: (0, 0)),
            pl.BlockSpec((N, D), lambda e: (0, 0)),
        ],
        out_specs=(
            pl.BlockSpec((1, N, CAP), lambda e: (e, 0, 0)),
            pl.BlockSpec((1, CAP, D), lambda e: (e, 0, 0)),
        ),
        out_shape=(
            jax.ShapeDtypeStruct((E, N, CAP), jnp.bfloat16),
            jax.ShapeDtypeStruct((E, CAP, D), jnp.bfloat16),
        ),
    )(selp, rank, y)

    h_tok = pl.pallas_call(
        _fc1_kernel,
        grid=(F1 // F1_BLK, E),
        in_specs=[
            pl.BlockSpec((1, CAP, D), lambda f, e: (e, 0, 0)),
            pl.BlockSpec((1, D, F1_BLK), lambda f, e: (e, 0, f)),
            pl.BlockSpec((1, F1_BLK), lambda f, e: (0, f)),
            pl.BlockSpec((1, N, CAP), lambda f, e: (e, 0, 0)),
        ],
        out_specs=pl.BlockSpec((N, F1_BLK), lambda f, e: (0, f)),
        out_shape=jax.ShapeDtypeStruct((N, F1), jnp.bfloat16),
        scratch_shapes=[pltpu.VMEM((N, F1_BLK), jnp.float32)],
    )(xe_all, W1, b1r, pt_all)

    xa = pl.pallas_call(
        _attn_kernel,
        grid=(HEADS // 2,),
        in_specs=[
            pl.BlockSpec((N, 2 * HD), lambda h: (0, (MLP // (2 * HD)) + h)),
            pl.BlockSpec((N, 2 * HD), lambda h: (0, ((MLP + D) // (2 * HD)) + h)),
            pl.BlockSpec((N, 2 * HD), lambda h: (0, ((MLP + 2 * D) // (2 * HD)) + h)),
        ],
        out_specs=pl.BlockSpec((N, 2 * HD), lambda h: (0, h)),
        out_shape=jax.ShapeDtypeStruct((N, D), jnp.bfloat16),
    )(h_tok, h_tok, h_tok)

    ye_all = pl.pallas_call(
        _gather2_kernel,
        grid=(E,),
        in_specs=[
            pl.BlockSpec((1, N, CAP), lambda e: (e, 0, 0)),
            pl.BlockSpec((N, MLP), lambda e: (0, 0)),
            pl.BlockSpec((N, D), lambda e: (0, 0)),
        ],
        out_specs=pl.BlockSpec((1, CAP, F2IN), lambda e: (e, 0, 0)),
        out_shape=jax.ShapeDtypeStruct((E, CAP, F2IN), jnp.bfloat16),
    )(pt_all, h_tok, xa)

    out_tok = pl.pallas_call(
        _fc2_kernel,
        grid=(2, E),
        in_specs=[
            pl.BlockSpec((1, CAP, F2IN), lambda o, e: (e, 0, 0)),
            pl.BlockSpec((1, F2IN, F2OUT // 2), lambda o, e: (e, 0, o)),
            pl.BlockSpec((1, F2OUT // 2), lambda o, e: (0, o)),
            pl.BlockSpec((N, E), lambda o, e: (0, 0)),
            pl.BlockSpec((1, N, CAP), lambda o, e: (e, 0, 0)),
        ],
        out_specs=pl.BlockSpec((N, F2OUT // 2), lambda o, e: (0, o)),
        out_shape=jax.ShapeDtypeStruct((N, F2OUT), jnp.float32),
    )(ye_all, W2, b2r, selp, pt_all)

    out = pl.pallas_call(
        _final_kernel,
        in_specs=[
            pl.BlockSpec((N, D), lambda: (0, 0)),
            pl.BlockSpec((N, F2OUT), lambda: (0, 0)),
        ],
        out_specs=pl.BlockSpec((N, D), lambda: (0, 0)),
        out_shape=jax.ShapeDtypeStruct((N, D), jnp.float32),
    )(x2, out_tok)

    return out[None]
